# baseline (speedup 1.0000x reference)
import functools
import jax
import jax.numpy as jnp
import numpy as np
from jax import lax
from jax.experimental import pallas as pl
from jax.experimental.pallas import tpu as pltpu
from jax.experimental.pallas import tpu_sc as plsc

_ROWS = 8192
_COLS = 2048
_LANES = 16
_NV = _COLS // _LANES
_NW = 32
_ROWS_PER_W = _ROWS // _NW
_RBLK = 8
_NBLK = _ROWS_PER_W // _RBLK
_NPAIR = _NBLK // 2
_G = _RBLK


def _scan_block(src, dst):
    @plsc.parallel_loop(0, _NV, carry=(jnp.float32(0),) * _G, unroll=8)
    def _loop(i, carries):
        off = i * _LANES
        svals = []
        for u in range(_G):
            v = src[u, pl.ds(off, _LANES)]
            svals.append(plsc.cumsum(v))
        new = []
        for u in range(_G):
            dst[u, pl.ds(off, _LANES)] = svals[u] + carries[u]
            new.append(carries[u] + svals[u][_LANES - 1])
        return tuple(new)


def _sc_body(x_hbm, out_hbm, in0, in1, ou0, ou1, si0, si1, so0, so1):
    wid = lax.axis_index("s") * 2 + lax.axis_index("c")
    base = wid * _ROWS_PER_W

    def in_slice(b):
        return x_hbm.at[pl.ds(base + b * _RBLK, _RBLK)]

    def out_slice(b):
        return out_hbm.at[pl.ds(base + b * _RBLK, _RBLK)]

    pltpu.async_copy(in_slice(0), in0, si0)

    def body(k, c):
        b0 = 2 * k
        pltpu.async_copy(in_slice(b0 + 1), in1, si1)
        pltpu.make_async_copy(in_slice(b0), in0, si0).wait()

        @pl.when(k > 0)
        def _():
            pltpu.make_async_copy(ou0, out_slice(b0), so0).wait()

        _scan_block(in0, ou0)
        pltpu.async_copy(ou0, out_slice(b0), so0)

        @pl.when(k < _NPAIR - 1)
        def _():
            pltpu.async_copy(in_slice(b0 + 2), in0, si0)

        pltpu.make_async_copy(in_slice(b0 + 1), in1, si1).wait()

        @pl.when(k > 0)
        def _():
            pltpu.make_async_copy(ou1, out_slice(b0 + 1), so1).wait()

        _scan_block(in1, ou1)
        pltpu.async_copy(ou1, out_slice(b0 + 1), so1)
        return c

    lax.fori_loop(0, _NPAIR, body, 0, unroll=1)
    pltpu.make_async_copy(ou0, out_slice(_NBLK - 2), so0).wait()
    pltpu.make_async_copy(ou1, out_slice(_NBLK - 1), so1).wait()


@jax.jit
def kernel(x):
    mesh = plsc.VectorSubcoreMesh(core_axis_name="c", subcore_axis_name="s")
    run = pl.kernel(
        _sc_body,
        out_type=jax.ShapeDtypeStruct((_ROWS, _COLS), jnp.float32),
        mesh=mesh,
        scratch_types=[
            pltpu.VMEM((_RBLK, _COLS), jnp.float32),
            pltpu.VMEM((_RBLK, _COLS), jnp.float32),
            pltpu.VMEM((_RBLK, _COLS), jnp.float32),
            pltpu.VMEM((_RBLK, _COLS), jnp.float32),
            pltpu.SemaphoreType.DMA,
            pltpu.SemaphoreType.DMA,
            pltpu.SemaphoreType.DMA,
            pltpu.SemaphoreType.DMA,
        ],
        compiler_params=pltpu.CompilerParams(needs_layout_passes=False, skip_device_barrier=True),
    )
    return run(x)


# speedup vs baseline: 1.0435x; 1.0435x over previous
import functools
import jax
import jax.numpy as jnp
import numpy as np
from jax import lax
from jax.experimental import pallas as pl
from jax.experimental.pallas import tpu as pltpu
from jax.experimental.pallas import tpu_sc as plsc

_ROWS = 8192
_COLS = 2048
_LANES = 16
_NV = _COLS // _LANES
_NW = 32
_ROWS_PER_W = _ROWS // _NW
_RBLK = 8
_NBLK = _ROWS_PER_W // _RBLK
_NRING = 4
_NITER = _NBLK // _NRING
_G = _RBLK


def _scan_block(src, dst):
    @plsc.parallel_loop(0, _NV, carry=(jnp.float32(0),) * _G, unroll=8)
    def _loop(i, carries):
        off = i * _LANES
        svals = []
        for u in range(_G):
            v = src[u, pl.ds(off, _LANES)]
            svals.append(plsc.cumsum(v))
        new = []
        for u in range(_G):
            dst[u, pl.ds(off, _LANES)] = svals[u] + carries[u]
            new.append(carries[u] + svals[u][_LANES - 1])
        return tuple(new)


def _sc_body(x_hbm, out_hbm, i0, i1, i2, i3, ou0, ou1,
             si0, si1, si2, si3, so0, so1):
    wid = lax.axis_index("s") * 2 + lax.axis_index("c")
    base = wid * _ROWS_PER_W
    ins = (i0, i1, i2, i3)
    isems = (si0, si1, si2, si3)
    outs = (ou0, ou1)
    osems = (so0, so1)

    def in_slice(b):
        return x_hbm.at[pl.ds(base + b * _RBLK, _RBLK)]

    def out_slice(b):
        return out_hbm.at[pl.ds(base + b * _RBLK, _RBLK)]

    for j in range(_NRING):
        pltpu.async_copy(in_slice(j), ins[j], isems[j])

    def body(k, c):
        for j in range(_NRING):
            b = _NRING * k + j
            p = j % 2
            pltpu.make_async_copy(in_slice(b), ins[j], isems[j]).wait()
            if j < 2:
                @pl.when(k > 0)
                def _():
                    pltpu.make_async_copy(outs[p], out_slice(b - 2),
                                          osems[p]).wait()
            else:
                pltpu.make_async_copy(outs[p], out_slice(b - 2),
                                      osems[p]).wait()
            _scan_block(ins[j], outs[p])
            pltpu.async_copy(outs[p], out_slice(b), osems[p])

            @pl.when(k < _NITER - 1)
            def _():
                pltpu.async_copy(in_slice(b + _NRING), ins[j], isems[j])
        return c

    lax.fori_loop(0, _NITER, body, 0, unroll=1)
    pltpu.make_async_copy(ou0, out_slice(_NBLK - 2), so0).wait()
    pltpu.make_async_copy(ou1, out_slice(_NBLK - 1), so1).wait()


@jax.jit
def kernel(x):
    mesh = plsc.VectorSubcoreMesh(core_axis_name="c", subcore_axis_name="s")
    run = pl.kernel(
        _sc_body,
        out_type=jax.ShapeDtypeStruct((_ROWS, _COLS), jnp.float32),
        mesh=mesh,
        scratch_types=[
            pltpu.VMEM((_RBLK, _COLS), jnp.float32),
            pltpu.VMEM((_RBLK, _COLS), jnp.float32),
            pltpu.VMEM((_RBLK, _COLS), jnp.float32),
            pltpu.VMEM((_RBLK, _COLS), jnp.float32),
            pltpu.VMEM((_RBLK, _COLS), jnp.float32),
            pltpu.VMEM((_RBLK, _COLS), jnp.float32),
            pltpu.SemaphoreType.DMA,
            pltpu.SemaphoreType.DMA,
            pltpu.SemaphoreType.DMA,
            pltpu.SemaphoreType.DMA,
            pltpu.SemaphoreType.DMA,
            pltpu.SemaphoreType.DMA,
        ],
        compiler_params=pltpu.CompilerParams(needs_layout_passes=False),
    )
    return run(x)
